# baseline (speedup 1.0000x reference)
import functools

import jax
import jax.numpy as jnp
from jax import lax
from jax.experimental import pallas as pl
from jax.experimental.pallas import tpu as pltpu
from jax.experimental.pallas import tpu_sc as plsc

N = 10000
NP = 10240
E = 320000
NG = 64
D_IN = 128
H1 = 16
H2 = 32

NC = 2
NS = 16
NW = NC * NS
EPW = E // NW
CE = 128
NCHUNK = 80
EPWP = NCHUNK * CE
NR = NP // NS

RB = 1000
NBLK = N // RB

def _sc_mesh():
  return plsc.VectorSubcoreMesh(
      core_axis_name="c", subcore_axis_name="s", num_cores=NC,
      num_subcores=NS)


@functools.cache
def _make_deg_kernel():
  @functools.partial(
      pl.kernel,
      out_type=jax.ShapeDtypeStruct((NC, NP), jnp.float32),
      mesh=_sc_mesh(),
      scratch_types=[
          pltpu.VMEM((NCHUNK, CE), jnp.int32),
          pltpu.VMEM((CE,), jnp.float32),
          pltpu.VMEM_SHARED((NP,), jnp.float32),
          pltpu.SemaphoreType.DMA,
          pltpu.SemaphoreType.DMA,
      ],
      compiler_params=pltpu.CompilerParams(use_tc_tiling_on_sc=False),
  )
  def deg_k(dst_hbm, z_hbm, out_hbm, didx, ones_v, acc, sm0, sm1):
    c = lax.axis_index("c")
    s = lax.axis_index("s")
    wid = s * NC + c
    pltpu.sync_copy(z_hbm.at[pl.ds(s * NR, NR)], acc.at[pl.ds(s * NR, NR)])
    pltpu.sync_copy(dst_hbm.at[wid], didx)
    for k in range(CE // 16):
      ones_v[pl.ds(k * 16, 16)] = jnp.ones((16,), jnp.float32)
    plsc.subcore_barrier()

    pltpu.async_copy(ones_v, acc.at[didx.at[0]], sm0, add=True)
    pltpu.async_copy(ones_v, acc.at[didx.at[1]], sm1, add=True)

    def chunk(q, carry):
      j = 2 * q
      pltpu.make_async_copy(ones_v, acc.at[didx.at[0]], sm0).wait()
      pltpu.async_copy(ones_v, acc.at[didx.at[j + 2]], sm0, add=True)
      pltpu.make_async_copy(ones_v, acc.at[didx.at[0]], sm1).wait()
      pltpu.async_copy(ones_v, acc.at[didx.at[j + 3]], sm1, add=True)
      return carry

    lax.fori_loop(0, (NCHUNK - 2) // 2, chunk, 0)
    pltpu.make_async_copy(ones_v, acc.at[didx.at[0]], sm0).wait()
    pltpu.make_async_copy(ones_v, acc.at[didx.at[0]], sm1).wait()
    plsc.subcore_barrier()
    pltpu.sync_copy(acc.at[pl.ds(s * NR, NR)],
                    out_hbm.at[c].at[pl.ds(s * NR, NR)])

  return deg_k


@functools.cache
def _make_edge_kernel(h, ce):
  nchunk = EPWP // ce
  assert nchunk % 4 == 0 and nchunk >= 8

  @functools.partial(
      pl.kernel,
      out_type=jax.ShapeDtypeStruct((NC, NP, h), jnp.float32),
      mesh=_sc_mesh(),
      scratch_types=[
          pltpu.VMEM((nchunk, ce), jnp.int32),
          pltpu.VMEM((nchunk, ce), jnp.int32),
          pltpu.VMEM((ce, h), jnp.float32),
          pltpu.VMEM((ce, h), jnp.float32),
          pltpu.VMEM((ce, h), jnp.float32),
          pltpu.VMEM((ce, h), jnp.float32),
          pltpu.SemaphoreType.DMA,
          pltpu.SemaphoreType.DMA,
          pltpu.SemaphoreType.DMA,
          pltpu.SemaphoreType.DMA,
          pltpu.SemaphoreType.DMA,
          pltpu.SemaphoreType.DMA,
          pltpu.SemaphoreType.DMA,
          pltpu.SemaphoreType.DMA,
          pltpu.VMEM_SHARED((NP, h), jnp.float32),
      ],
      compiler_params=pltpu.CompilerParams(use_tc_tiling_on_sc=False),
  )
  def edge_k(src_hbm, dst_hbm, g_hbm, z_hbm, out_hbm, sidx, didx, r0, r1, r2,
             r3, g0, g1_, g2_, g3, s0, s1, s2, s3, acc):
    c = lax.axis_index("c")
    s = lax.axis_index("s")
    wid = s * NC + c
    pltpu.sync_copy(z_hbm.at[pl.ds(s * NR, NR)], acc.at[pl.ds(s * NR, NR)])
    pltpu.sync_copy(src_hbm.at[wid], sidx)
    pltpu.sync_copy(dst_hbm.at[wid], didx)
    plsc.subcore_barrier()

    bufs = (r0, r1, r2, r3)
    gsems = (g0, g1_, g2_, g3)
    ssems = (s0, s1, s2, s3)

    def gather(j, b):
      pltpu.async_copy(g_hbm.at[sidx.at[j]], bufs[b], gsems[b])

    def gather_wait(j, b):
      pltpu.make_async_copy(g_hbm.at[sidx.at[j]], bufs[b], gsems[b]).wait()

    def scatter(j, b):
      pltpu.async_copy(bufs[b], acc.at[didx.at[j]], ssems[b], add=True)

    def scatter_wait(b):
      pltpu.make_async_copy(bufs[b], acc.at[didx.at[0]], ssems[b]).wait()

    for b in range(3):
      gather(b, b)
    for b in range(4):
      gather_wait(b, b)
      scatter(b, b)
      bn = (b + 3) % 4
      if b > 0:
        scatter_wait(bn)
      gather(b + 3, bn)

    def outer(q, carry):
      j0 = 4 * q
      for b in range(4):
        j = j0 + b
        gather_wait(j, b)
        scatter(j, b)
        bn = (b + 3) % 4

        @pl.when(j + 3 < nchunk)
        def _():
          scatter_wait(bn)
          gather(j + 3, bn)
      return carry

    lax.fori_loop(1, nchunk // 4, outer, 0)
    for b in range(4):
      scatter_wait(b)
    plsc.subcore_barrier()
    pltpu.sync_copy(acc.at[pl.ds(s * NR, NR)],
                    out_hbm.at[c].at[pl.ds(s * NR, NR)])

  return edge_k


def _deg_kernel(dst, z1):
  return _make_deg_kernel()(dst.reshape(NW, NCHUNK, CE), z1)


CE1 = 128
CE2 = 128


def _edge_kernel_h1(src, dst, g, z):
  return _make_edge_kernel(H1, CE1)(
      src.reshape(NW, EPWP // CE1, CE1), dst.reshape(NW, EPWP // CE1, CE1),
      g, z)


def _edge_kernel_h2(src, dst, g, z):
  return _make_edge_kernel(H2, CE2)(
      src.reshape(NW, EPWP // CE2, CE2), dst.reshape(NW, EPWP // CE2, CE2),
      g, z)


def _mm1s(xp, W1, deg0, deg1):
  def body(x_ref, w_ref, d0_ref, d1_ref, g_ref, dinv_ref):
    dinv = lax.rsqrt(d0_ref[...] + d1_ref[...] + 1.0)
    dinv_ref[...] = dinv
    g_ref[...] = dinv * jnp.dot(x_ref[...], w_ref[...],
                                preferred_element_type=jnp.float32)

  return pl.pallas_call(
      body,
      grid=(NBLK,),
      in_specs=[
          pl.BlockSpec((RB, D_IN), lambda i: (i, 0)),
          pl.BlockSpec((D_IN, H1), lambda i: (0, 0)),
          pl.BlockSpec((RB, 1), lambda i: (i, 0)),
          pl.BlockSpec((RB, 1), lambda i: (i, 0)),
      ],
      out_specs=[
          pl.BlockSpec((RB, H1), lambda i: (i, 0)),
          pl.BlockSpec((RB, 1), lambda i: (i, 0)),
      ],
      out_shape=[
          jax.ShapeDtypeStruct((N, H1), jnp.float32),
          jax.ShapeDtypeStruct((N, 1), jnp.float32),
      ],
  )(xp, W1, deg0, deg1)


def _mid(acc_a, acc_b, g1, dinv, b1, W2):
  def body(aa_ref, ab_ref, g_ref, d_ref, b_ref, w_ref, o_ref):
    a1 = jnp.maximum(
        d_ref[...] * (aa_ref[...] + ab_ref[...] + g_ref[...]) + b_ref[...],
        0.0)
    o_ref[...] = d_ref[...] * jnp.dot(a1, w_ref[...],
                                      preferred_element_type=jnp.float32)

  return pl.pallas_call(
      body,
      grid=(NBLK,),
      in_specs=[
          pl.BlockSpec((RB, H1), lambda i: (i, 0)),
          pl.BlockSpec((RB, H1), lambda i: (i, 0)),
          pl.BlockSpec((RB, H1), lambda i: (i, 0)),
          pl.BlockSpec((RB, 1), lambda i: (i, 0)),
          pl.BlockSpec((1, H1), lambda i: (0, 0)),
          pl.BlockSpec((H1, H2), lambda i: (0, 0)),
      ],
      out_specs=pl.BlockSpec((RB, H2), lambda i: (i, 0)),
      out_shape=jax.ShapeDtypeStruct((N, H2), jnp.float32),
  )(acc_a, acc_b, g1, dinv, b1, W2)


def _final(acc_a, acc_b, g2, dinv, b2, bm, Wl, bl):
  def body(aa_ref, ab_ref, g_ref, d_ref, b_ref, bm_ref, wl_ref, bl_ref,
           o_ref, s_scr, c_scr):
    i = pl.program_id(0)

    @pl.when(i == 0)
    def _():
      s_scr[...] = jnp.zeros_like(s_scr)
      c_scr[...] = jnp.zeros_like(c_scr)

    a2 = jnp.maximum(
        d_ref[...] * (aa_ref[...] + ab_ref[...] + g_ref[...]) + b_ref[...],
        0.0)
    oh = (lax.broadcasted_iota(jnp.int32, (RB, NG), 1) ==
          bm_ref[...]).astype(jnp.float32)
    dnums = (((0,), (0,)), ((), ()))
    s_scr[...] += lax.dot_general(oh, a2, dnums,
                                  preferred_element_type=jnp.float32)
    c_scr[...] += lax.dot_general(oh, jnp.ones((RB, 1), jnp.float32), dnums,
                                  preferred_element_type=jnp.float32)

    @pl.when(i == NBLK - 1)
    def _():
      pooled = s_scr[...] / jnp.maximum(c_scr[...], 1.0)
      o_ref[...] = jnp.maximum(
          jnp.dot(pooled, wl_ref[...], preferred_element_type=jnp.float32) +
          bl_ref[...], 0.0)

  return pl.pallas_call(
      body,
      grid=(NBLK,),
      in_specs=[
          pl.BlockSpec((RB, H2), lambda i: (i, 0)),
          pl.BlockSpec((RB, H2), lambda i: (i, 0)),
          pl.BlockSpec((RB, H2), lambda i: (i, 0)),
          pl.BlockSpec((RB, 1), lambda i: (i, 0)),
          pl.BlockSpec((1, H2), lambda i: (0, 0)),
          pl.BlockSpec((RB, 1), lambda i: (i, 0)),
          pl.BlockSpec((H2, H2), lambda i: (0, 0)),
          pl.BlockSpec((1, H2), lambda i: (0, 0)),
      ],
      out_specs=pl.BlockSpec((NG, H2), lambda i: (0, 0)),
      out_shape=jax.ShapeDtypeStruct((NG, H2), jnp.float32),
      scratch_shapes=[
          pltpu.VMEM((NG, H2), jnp.float32),
          pltpu.VMEM((NG, 1), jnp.float32),
      ],
      compiler_params=pltpu.CompilerParams(
          dimension_semantics=("arbitrary",)),
  )(acc_a, acc_b, g2, dinv, b2, bm, Wl, bl)


def kernel(x, edge_index, edge_attr, batch_mask, W1, b1, W2, b2, Wl, bl):
  del edge_attr
  f32 = jnp.float32

  pad_dst = N + (jnp.arange(EPWP - EPW, dtype=jnp.int32) % (NP - N))
  src = jnp.concatenate(
      [edge_index[0].reshape(NW, EPW),
       jnp.zeros((NW, EPWP - EPW), jnp.int32)], axis=1)
  dst = jnp.concatenate(
      [edge_index[1].reshape(NW, EPW),
       jnp.broadcast_to(pad_dst, (NW, EPWP - EPW))], axis=1)
  bm = batch_mask.reshape(N, 1)
  z1 = jnp.zeros((NP,), f32)
  z16 = jnp.zeros((NP, H1), f32)
  z32 = jnp.zeros((NP, H2), f32)

  deg = _deg_kernel(dst, z1)
  g1, dinv = _mm1s(x, W1, deg[0, :N].reshape(N, 1), deg[1, :N].reshape(N, 1))
  acc1 = _edge_kernel_h1(src, dst, g1, z16)
  g2 = _mid(acc1[0, :N], acc1[1, :N], g1, dinv, b1.reshape(1, H1), W2)
  acc2 = _edge_kernel_h2(src, dst, g2, z32)
  out = _final(acc2[0, :N], acc2[1, :N], g2, dinv, b2.reshape(1, H2), bm, Wl,
               bl.reshape(1, H2))
  return out


# speedup vs baseline: 1.5317x; 1.5317x over previous
import functools

import jax
import jax.numpy as jnp
from jax import lax
from jax.experimental import pallas as pl
from jax.experimental.pallas import tpu as pltpu
from jax.experimental.pallas import tpu_sc as plsc

N = 10000
NP = 10240
E = 320000
NG = 64
D_IN = 128
H1 = 16
H2 = 32

NC = 2
NS = 16
NW = NC * NS
EPW = E // NW
CE = 128
NCHUNK = 80
EPWP = NCHUNK * CE
NR = NP // NS

RB = 1024
NBLK = NP // RB

def _sc_mesh():
  return plsc.VectorSubcoreMesh(
      core_axis_name="c", subcore_axis_name="s", num_cores=NC,
      num_subcores=NS)


@functools.cache
def _make_deg_kernel():
  @functools.partial(
      pl.kernel,
      out_type=jax.ShapeDtypeStruct((NC, NP), jnp.float32),
      mesh=_sc_mesh(),
      scratch_types=[
          pltpu.VMEM((NCHUNK, CE), jnp.int32),
          pltpu.VMEM((CE,), jnp.float32),
          pltpu.VMEM_SHARED((NP,), jnp.float32),
          pltpu.SemaphoreType.DMA,
          pltpu.SemaphoreType.DMA,
      ],
      compiler_params=pltpu.CompilerParams(use_tc_tiling_on_sc=False),
  )
  def deg_k(dst_hbm, z_hbm, out_hbm, didx, ones_v, acc, sm0, sm1):
    c = lax.axis_index("c")
    s = lax.axis_index("s")
    wid = s * NC + c
    pltpu.sync_copy(z_hbm.at[pl.ds(s * NR, NR)], acc.at[pl.ds(s * NR, NR)])
    pltpu.sync_copy(dst_hbm.at[wid], didx)
    for k in range(CE // 16):
      ones_v[pl.ds(k * 16, 16)] = jnp.ones((16,), jnp.float32)
    plsc.subcore_barrier()

    pltpu.async_copy(ones_v, acc.at[didx.at[0]], sm0, add=True)
    pltpu.async_copy(ones_v, acc.at[didx.at[1]], sm1, add=True)

    def chunk(q, carry):
      j = 2 * q
      pltpu.make_async_copy(ones_v, acc.at[didx.at[0]], sm0).wait()
      pltpu.async_copy(ones_v, acc.at[didx.at[j + 2]], sm0, add=True)
      pltpu.make_async_copy(ones_v, acc.at[didx.at[0]], sm1).wait()
      pltpu.async_copy(ones_v, acc.at[didx.at[j + 3]], sm1, add=True)
      return carry

    lax.fori_loop(0, (NCHUNK - 2) // 2, chunk, 0)
    pltpu.make_async_copy(ones_v, acc.at[didx.at[0]], sm0).wait()
    pltpu.make_async_copy(ones_v, acc.at[didx.at[0]], sm1).wait()
    plsc.subcore_barrier()
    pltpu.sync_copy(acc.at[pl.ds(s * NR, NR)],
                    out_hbm.at[c].at[pl.ds(s * NR, NR)])

  return deg_k


@functools.cache
def _make_edge_kernel(h, ce):
  nchunk = EPWP // ce
  assert nchunk % 4 == 0 and nchunk >= 8

  @functools.partial(
      pl.kernel,
      out_type=jax.ShapeDtypeStruct((NC, NP, h), jnp.float32),
      mesh=_sc_mesh(),
      scratch_types=[
          pltpu.VMEM((nchunk, ce), jnp.int32),
          pltpu.VMEM((nchunk, ce), jnp.int32),
          pltpu.VMEM((ce, h), jnp.float32),
          pltpu.VMEM((ce, h), jnp.float32),
          pltpu.VMEM((ce, h), jnp.float32),
          pltpu.VMEM((ce, h), jnp.float32),
          pltpu.SemaphoreType.DMA,
          pltpu.SemaphoreType.DMA,
          pltpu.SemaphoreType.DMA,
          pltpu.SemaphoreType.DMA,
          pltpu.SemaphoreType.DMA,
          pltpu.SemaphoreType.DMA,
          pltpu.SemaphoreType.DMA,
          pltpu.SemaphoreType.DMA,
          pltpu.VMEM_SHARED((NP, h), jnp.float32),
      ],
      compiler_params=pltpu.CompilerParams(use_tc_tiling_on_sc=False),
  )
  def edge_k(src_hbm, dst_hbm, g_hbm, z_hbm, out_hbm, sidx, didx, r0, r1, r2,
             r3, g0, g1_, g2_, g3, s0, s1, s2, s3, acc):
    c = lax.axis_index("c")
    s = lax.axis_index("s")
    wid = s * NC + c
    pltpu.sync_copy(z_hbm.at[pl.ds(s * NR, NR)], acc.at[pl.ds(s * NR, NR)])
    pltpu.sync_copy(src_hbm.at[wid], sidx)
    pltpu.sync_copy(dst_hbm.at[wid], didx)
    plsc.subcore_barrier()

    bufs = (r0, r1, r2, r3)
    gsems = (g0, g1_, g2_, g3)
    ssems = (s0, s1, s2, s3)

    def gather(j, b):
      pltpu.async_copy(g_hbm.at[sidx.at[j]], bufs[b], gsems[b])

    def gather_wait(j, b):
      pltpu.make_async_copy(g_hbm.at[sidx.at[j]], bufs[b], gsems[b]).wait()

    def scatter(j, b):
      pltpu.async_copy(bufs[b], acc.at[didx.at[j]], ssems[b], add=True)

    def scatter_wait(b):
      pltpu.make_async_copy(bufs[b], acc.at[didx.at[0]], ssems[b]).wait()

    for b in range(3):
      gather(b, b)
    for b in range(4):
      gather_wait(b, b)
      scatter(b, b)
      bn = (b + 3) % 4
      if b > 0:
        scatter_wait(bn)
      gather(b + 3, bn)

    def outer(q, carry):
      j0 = 4 * q
      for b in range(4):
        j = j0 + b
        gather_wait(j, b)
        scatter(j, b)
        bn = (b + 3) % 4

        @pl.when(j + 3 < nchunk)
        def _():
          scatter_wait(bn)
          gather(j + 3, bn)
      return carry

    lax.fori_loop(1, nchunk // 4, outer, 0)
    for b in range(4):
      scatter_wait(b)
    plsc.subcore_barrier()
    pltpu.sync_copy(acc.at[pl.ds(s * NR, NR)],
                    out_hbm.at[c].at[pl.ds(s * NR, NR)])

  return edge_k


def _deg_kernel(dst, z1):
  return _make_deg_kernel()(dst.reshape(NW, NCHUNK, CE), z1)


CE1 = 128
CE2 = 128


def _edge_kernel_h1(src, dst, g, z):
  return _make_edge_kernel(H1, CE1)(
      src.reshape(NW, EPWP // CE1, CE1), dst.reshape(NW, EPWP // CE1, CE1),
      g, z)


def _edge_kernel_h2(src, dst, g, z):
  return _make_edge_kernel(H2, CE2)(
      src.reshape(NW, EPWP // CE2, CE2), dst.reshape(NW, EPWP // CE2, CE2),
      g, z)


def _mm1s(xp, W1, deg0, deg1):
  def body(x_ref, w_ref, d0_ref, d1_ref, g_ref, dinv_ref):
    dinv = lax.rsqrt(d0_ref[...] + d1_ref[...] + 1.0)
    dinv_ref[...] = dinv
    g_ref[...] = dinv * jnp.dot(x_ref[...], w_ref[...],
                                preferred_element_type=jnp.float32)

  return pl.pallas_call(
      body,
      grid=(NBLK,),
      in_specs=[
          pl.BlockSpec((RB, D_IN), lambda i: (i, 0)),
          pl.BlockSpec((D_IN, H1), lambda i: (0, 0)),
          pl.BlockSpec((RB, 1), lambda i: (i, 0)),
          pl.BlockSpec((RB, 1), lambda i: (i, 0)),
      ],
      out_specs=[
          pl.BlockSpec((RB, H1), lambda i: (i, 0)),
          pl.BlockSpec((RB, 1), lambda i: (i, 0)),
      ],
      out_shape=[
          jax.ShapeDtypeStruct((NP, H1), jnp.float32),
          jax.ShapeDtypeStruct((NP, 1), jnp.float32),
      ],
  )(xp, W1, deg0, deg1)


def _mid(acc_a, acc_b, g1, dinv, b1, W2):
  def body(aa_ref, ab_ref, g_ref, d_ref, b_ref, w_ref, o_ref):
    a1 = jnp.maximum(
        d_ref[...] * (aa_ref[...] + ab_ref[...] + g_ref[...]) + b_ref[...],
        0.0)
    o_ref[...] = d_ref[...] * jnp.dot(a1, w_ref[...],
                                      preferred_element_type=jnp.float32)

  return pl.pallas_call(
      body,
      grid=(NBLK,),
      in_specs=[
          pl.BlockSpec((RB, H1), lambda i: (i, 0)),
          pl.BlockSpec((RB, H1), lambda i: (i, 0)),
          pl.BlockSpec((RB, H1), lambda i: (i, 0)),
          pl.BlockSpec((RB, 1), lambda i: (i, 0)),
          pl.BlockSpec((1, H1), lambda i: (0, 0)),
          pl.BlockSpec((H1, H2), lambda i: (0, 0)),
      ],
      out_specs=pl.BlockSpec((RB, H2), lambda i: (i, 0)),
      out_shape=jax.ShapeDtypeStruct((NP, H2), jnp.float32),
  )(acc_a, acc_b, g1, dinv, b1, W2)


def _final(acc_a, acc_b, g2, dinv, b2, bm, Wl, bl):
  def body(aa_ref, ab_ref, g_ref, d_ref, b_ref, bm_ref, wl_ref, bl_ref,
           o_ref, s_scr, c_scr):
    i = pl.program_id(0)

    @pl.when(i == 0)
    def _():
      s_scr[...] = jnp.zeros_like(s_scr)
      c_scr[...] = jnp.zeros_like(c_scr)

    a2 = jnp.maximum(
        d_ref[...] * (aa_ref[...] + ab_ref[...] + g_ref[...]) + b_ref[...],
        0.0)
    oh = (lax.broadcasted_iota(jnp.int32, (NG, RB), 0) ==
          bm_ref[...]).astype(jnp.float32)
    s_scr[...] += jnp.dot(oh, a2, preferred_element_type=jnp.float32)
    c_scr[...] += jnp.sum(oh, axis=1, keepdims=True)

    @pl.when(i == NBLK - 1)
    def _():
      pooled = s_scr[...] / jnp.maximum(c_scr[...], 1.0)
      o_ref[...] = jnp.maximum(
          jnp.dot(pooled, wl_ref[...], preferred_element_type=jnp.float32) +
          bl_ref[...], 0.0)

  return pl.pallas_call(
      body,
      grid=(NBLK,),
      in_specs=[
          pl.BlockSpec((RB, H2), lambda i: (i, 0)),
          pl.BlockSpec((RB, H2), lambda i: (i, 0)),
          pl.BlockSpec((RB, H2), lambda i: (i, 0)),
          pl.BlockSpec((RB, 1), lambda i: (i, 0)),
          pl.BlockSpec((1, H2), lambda i: (0, 0)),
          pl.BlockSpec((1, RB), lambda i: (0, i)),
          pl.BlockSpec((H2, H2), lambda i: (0, 0)),
          pl.BlockSpec((1, H2), lambda i: (0, 0)),
      ],
      out_specs=pl.BlockSpec((NG, H2), lambda i: (0, 0)),
      out_shape=jax.ShapeDtypeStruct((NG, H2), jnp.float32),
      scratch_shapes=[
          pltpu.VMEM((NG, H2), jnp.float32),
          pltpu.VMEM((NG, 1), jnp.float32),
      ],
      compiler_params=pltpu.CompilerParams(
          dimension_semantics=("arbitrary",)),
  )(acc_a, acc_b, g2, dinv, b2, bm, Wl, bl)


def kernel(x, edge_index, edge_attr, batch_mask, W1, b1, W2, b2, Wl, bl):
  del edge_attr
  f32 = jnp.float32

  xp = jnp.zeros((NP, D_IN), f32).at[:N].set(x)
  pad_ids = N + (jnp.arange(EPWP - EPW, dtype=jnp.int32) % (NP - N))
  pad_blk = jnp.broadcast_to(pad_ids, (NW, EPWP - EPW))
  src = jnp.concatenate(
      [edge_index[0].reshape(NW, EPW), pad_blk], axis=1)
  dst = jnp.concatenate(
      [edge_index[1].reshape(NW, EPW), pad_blk], axis=1)
  bm = jnp.full((1, NP), NG, jnp.int32).at[0, :N].set(batch_mask)
  z1 = jnp.zeros((NP,), f32)
  z16 = jnp.zeros((NP, H1), f32)
  z32 = jnp.zeros((NP, H2), f32)

  deg = _deg_kernel(dst, z1)
  g1, dinv = _mm1s(xp, W1, deg[0].reshape(NP, 1), deg[1].reshape(NP, 1))
  acc1 = _edge_kernel_h1(src, dst, g1, z16)
  g2 = _mid(acc1[0], acc1[1], g1, dinv, b1.reshape(1, H1), W2)
  acc2 = _edge_kernel_h2(src, dst, g2, z32)
  out = _final(acc2[0], acc2[1], g2, dinv, b2.reshape(1, H2), bm, Wl,
               bl.reshape(1, H2))
  return out
